# baseline (speedup 1.0000x reference)
import functools

import jax
import jax.numpy as jnp
from jax.experimental import pallas as pl
from jax.experimental.pallas import tpu as pltpu

_CHUNK = 128
_DAP = 144
_PK = _DAP // 4


def _rec_kernel(
    uid_ref, iid_ref,
    uf_ref, cf_ref, if_ref,
    u_tab_ref, i_tab_ref,
    red_ref,
    w_u_ref, w_c_ref, w_i_ref,
    out_ref,
):
    f32 = jnp.float32
    tb = uf_ref.shape[0]
    nv = u_tab_ref.shape[1]
    ngrp = nv // 128

    def head(w_ref, feat_ref):
        return jax.lax.dot_general(
            w_ref[...], feat_ref[...],
            dimension_numbers=(((1,), (1,)), ((), ())),
            preferred_element_type=f32)

    uh = head(w_u_ref, uf_ref) + head(w_c_ref, cf_ref)
    ih = head(w_i_ref, if_ref)

    tabs = [[t[:, g * 128:(g + 1) * 128] for g in range(ngrp)]
            for t in (u_tab_ref[...], i_tab_ref[...])]

    def gather_chunk(tab_grps, ids_row):
        lo = jnp.broadcast_to(ids_row & 127, (_PK, _CHUNK))
        grps = [jnp.take_along_axis(tab_grps[g], lo, axis=1)
                for g in range(ngrp)]
        bit = 128
        while len(grps) > 1:
            m = (ids_row & bit) != 0
            grps = [jnp.where(m, b, a) for a, b in zip(grps[::2], grps[1::2])]
            bit <<= 1
        acc = grps[0]
        return jnp.concatenate(
            [((acc << (24 - 8 * j)) >> 24).astype(f32) for j in range(4)],
            axis=0)

    for c in range(tb // _CHUNK):
        sl = slice(c * _CHUNK, (c + 1) * _CHUNK)
        ug = gather_chunk(tabs[0], uid_ref[0][:, sl])
        ig = gather_chunk(tabs[1], iid_ref[0][:, sl])
        prod = (ug + uh[:, sl]) * (ig + ih[:, sl])
        out_ref[:, sl] = jax.lax.dot_general(
            red_ref[...], prod,
            dimension_numbers=(((1,), (0,)), ((), ())),
            preferred_element_type=f32)


def _quant_table(aug_f32):
    maxabs = jnp.max(jnp.abs(aug_f32), axis=1, keepdims=True)
    scale = jnp.exp2(jnp.ceil(jnp.log2(jnp.maximum(maxabs, 1e-30) / 127.0)))
    q = jnp.clip(jnp.round(aug_f32 / scale), -127, 127).astype(jnp.int32)
    b = [q[j * _PK:(j + 1) * _PK, :] & 255 for j in range(4)]
    packed = b[0] | (b[1] << 8) | (b[2] << 16) | (b[3] << 24)
    return packed.astype(jnp.int32), scale


@functools.partial(jax.jit, static_argnames=("tile_b",))
def _forward(
    user_ids, item_ids,
    user_table, item_table, user_bias_table, item_bias_table,
    user_feat, w_user, b_user,
    ctx_feat, w_ctx, b_ctx,
    item_feat, w_item, b_item,
    tile_b: int = 8192,
):
    f32 = jnp.float32
    B = user_ids.shape[0]
    NU, D = user_table.shape
    NI = item_table.shape[0]
    FU, FC, FI = user_feat.shape[1], ctx_feat.shape[1], item_feat.shape[1]

    b_uc_col = (b_user.reshape(-1) + b_ctx.reshape(-1)).astype(f32).reshape(D, 1)
    u_aug = jnp.zeros((_DAP, NU), f32)
    u_aug = u_aug.at[:D, :].set(user_table.astype(f32).T + b_uc_col)
    u_aug = u_aug.at[D, :].set(user_bias_table.astype(f32))
    u_aug = u_aug.at[D + 1, :].set(1.0)

    b_it_col = b_item.reshape(-1).astype(f32).reshape(D, 1)
    i_aug = jnp.zeros((_DAP, NI), f32)
    i_aug = i_aug.at[:D, :].set(item_table.astype(f32).T + b_it_col)
    i_aug = i_aug.at[D, :].set(1.0)
    i_aug = i_aug.at[D + 1, :].set(item_bias_table.astype(f32))

    u_pack, u_sc = _quant_table(u_aug)
    i_pack, i_sc = _quant_table(i_aug)
    red = (u_sc * i_sc).reshape(1, _DAP)

    def _aug_w(w, sc):
        wt = jnp.pad(w.astype(f32).T, ((0, _DAP - D), (0, 0)))
        return wt / sc

    w_u, w_c = _aug_w(w_user, u_sc), _aug_w(w_ctx, u_sc)
    w_i = _aug_w(w_item, i_sc)

    num_tiles = B // tile_b
    uid = user_ids.astype(jnp.int32).reshape(num_tiles, 1, tile_b)
    iid = item_ids.astype(jnp.int32).reshape(num_tiles, 1, tile_b)

    uid_spec = pl.BlockSpec((1, 1, tile_b), lambda b: (b, 0, 0))
    feat_spec = lambda cols: pl.BlockSpec((tile_b, cols), lambda b: (b, 0))
    const_spec = lambda shape: pl.BlockSpec(shape, lambda b: (0, 0))

    flops = 2 * B * _DAP * (FU + FC + FI + 1)
    bytes_accessed = (
        2 * B * 4 + B * (FU + FC + FI) * 4
        + (NU + NI) * _PK * 4 + 3 * 128 * _DAP * 4 + B * 4)

    out = pl.pallas_call(
        _rec_kernel,
        out_shape=jax.ShapeDtypeStruct((1, B), f32),
        grid=(num_tiles,),
        in_specs=[
            uid_spec, uid_spec,
            feat_spec(FU), feat_spec(FC), feat_spec(FI),
            const_spec((_PK, NU)), const_spec((_PK, NI)),
            const_spec((1, _DAP)),
            const_spec((_DAP, FU)), const_spec((_DAP, FC)), const_spec((_DAP, FI)),
        ],
        out_specs=pl.BlockSpec((1, tile_b), lambda b: (0, b)),
        compiler_params=pltpu.CompilerParams(
            dimension_semantics=("parallel",),
            vmem_limit_bytes=64 * 1024 * 1024,
        ),
        cost_estimate=pl.CostEstimate(
            flops=flops, transcendentals=0, bytes_accessed=bytes_accessed),
    )(
        uid, iid,
        user_feat, ctx_feat, item_feat,
        u_pack, i_pack,
        red,
        w_u, w_c, w_i,
    )
    return out[0]


def kernel(user_ids, item_ids, user_table, item_table, user_bias_table,
           item_bias_table, user_feat, w_user, b_user, ctx_feat, w_ctx, b_ctx,
           item_feat, w_item, b_item):
    return _forward(
        user_ids, item_ids,
        user_table, item_table, user_bias_table, item_bias_table,
        user_feat, w_user, b_user,
        ctx_feat, w_ctx, b_ctx,
        item_feat, w_item, b_item,
        tile_b=8192,
    )
